# Initial kernel scaffold; baseline (speedup 1.0000x reference)
import jax
import jax.numpy as jnp
from jax.experimental import pallas as pl


def kernel(feat, loc, edge_index, mid, W_fc2, W_fcd, W_w1, W_w2, vec_a, fin_table, boundaries):
    raise NotImplementedError("write your pallas kernel here")


# speedup vs baseline: 4.0020x; 4.0020x over previous
import functools

import jax
import jax.numpy as jnp
from jax import lax
from jax.experimental import pallas as pl
from jax.experimental.pallas import tpu as pltpu, tpu_sc as plsc

N = 10000
E = 160000
F = 256
FH = 128
NT = 16
EPT = E // NT
C = 80
NCHUNK = EPT // C
N_PAD = 10240
ROWS_PT = N_PAD // NT
DEGP = NT * 640

_SC_PARAMS = pltpu.CompilerParams(needs_layout_passes=False)


def _tc_dense(feat, w_pair, v8):
    def body(f_ref, w_ref, v_ref, h2_ref, st_ref):
        f = f_ref[...]
        w = w_ref[0]
        h2_ref[...] = lax.dot_general(
            f, w, (((1,), (1,)), ((), ())),
            precision=lax.Precision.HIGHEST,
            preferred_element_type=jnp.float32)
        st_ref[...] = lax.dot_general(
            f, v_ref[...], (((1,), (0,)), ((), ())),
            precision=lax.Precision.HIGHEST,
            preferred_element_type=jnp.float32)

    return pl.pallas_call(
        body,
        grid=(2, 10),
        in_specs=[
            pl.BlockSpec((1000, 256), lambda j, i: (i, 0)),
            pl.BlockSpec((1, 128, 256), lambda j, i: (j, 0, 0)),
            pl.BlockSpec((256, 8), lambda j, i: (0, 0)),
        ],
        out_specs=[
            pl.BlockSpec((1000, 128), lambda j, i: (j * 10 + i, 0)),
            pl.BlockSpec((1000, 8), lambda j, i: (i, 0)),
        ],
        out_shape=[
            jax.ShapeDtypeStruct((2 * N, FH), jnp.float32),
            jax.ShapeDtypeStruct((N, 8), jnp.float32),
        ],
    )(feat, w_pair, v8)


def _sc_coeffs(srcs, dsts, mids, sd_h, sm_h, ss_h, lx_h, ly_h, sdist_h,
               b2_h, p2_h):
    mesh = plsc.VectorSubcoreMesh(core_axis_name="c", subcore_axis_name="s")

    @functools.partial(
        pl.kernel,
        out_type=[jax.ShapeDtypeStruct((E,), jnp.float32),
                  jax.ShapeDtypeStruct((E,), jnp.float32)],
        mesh=mesh,
        compiler_params=_SC_PARAMS,
        scratch_types=[
            pltpu.VMEM_SHARED((DEGP,), jnp.float32),
            pltpu.VMEM((N,), jnp.float32),
            pltpu.VMEM((N,), jnp.float32),
            pltpu.VMEM((N,), jnp.float32),
            pltpu.VMEM((DEGP,), jnp.float32),
            pltpu.VMEM((N,), jnp.float32),
            pltpu.VMEM((N,), jnp.float32),
            pltpu.VMEM((128,), jnp.float32),
            pltpu.VMEM((128,), jnp.float32),
            pltpu.VMEM((128,), jnp.float32),
            pltpu.VMEM((C,), jnp.int32),
            pltpu.VMEM((C,), jnp.int32),
            pltpu.VMEM((C,), jnp.int32),
            pltpu.VMEM((C,), jnp.float32),
            pltpu.VMEM((C,), jnp.float32),
            pltpu.VMEM((C,), jnp.float32),
            pltpu.VMEM((640,), jnp.float32),
        ],
    )
    def k(src_hbm, dst_hbm, mid_hbm, sd_hbm, sm_hbm, ss_hbm, lx_hbm,
          ly_hbm, sdist_hbm, b2_hbm, p2_hbm, a_hbm, b_hbm,
          deg, sd, sm, ss, d0, lxr, lyr, sdt, b2r, p2r,
          srcc, dstc, midc, av, bv, ones, z640):
        c = lax.axis_index("c")
        s = lax.axis_index("s")
        zero16 = jnp.zeros((16,), jnp.float32)
        one16 = jnp.full((16,), 1.0, jnp.float32)
        sync = pltpu.sync_copy

        @pl.when(c == 0)
        def _core0():
            for i in range(C // 16):
                ones[pl.ds(i * 16, 16)] = one16

            def zb(i, _):
                z640[pl.ds(i * 16, 16)] = zero16
                return 0

            lax.fori_loop(0, 640 // 16, zb, 0)
            sync(z640, deg.at[pl.ds(s * 640, 640)])
            plsc.subcore_barrier()

            def degk(kk, _):
                off = s * EPT + kk * C
                sync(src_hbm.at[pl.ds(off, C)], srcc)
                sync(ones, deg.at[srcc], add=True)
                return 0

            lax.fori_loop(0, NCHUNK, degk, 0)
            plsc.subcore_barrier()

            sync(sd_hbm, sd)
            sync(sm_hbm, sm)
            sync(ss_hbm, ss)
            sync(lx_hbm, lxr)
            sync(ly_hbm, lyr)
            sync(sdist_hbm, sdt)
            sync(b2_hbm, b2r)
            sync(p2_hbm, p2r)
            sync(deg, d0)

            def rsq(i, _):
                sl = pl.ds(i * 16, 16)
                x = jnp.maximum(d0[sl], 1.0)
                kk = jnp.zeros((16,), jnp.int32)
                for t in range(1, 19):
                    kk = kk + jnp.where(x >= float(2 ** t), 1, 0)
                y = plsc.load_gather(p2r, [kk])
                for _ in range(5):
                    y = y * (1.5 - 0.5 * x * y * y)
                d0[sl] = y
                return 0

            lax.fori_loop(0, DEGP // 16, rsq, 0)

            def edgek(kk, _):
                off = s * EPT + kk * C
                sync(src_hbm.at[pl.ds(off, C)], srcc)
                sync(dst_hbm.at[pl.ds(off, C)], dstc)
                sync(mid_hbm.at[pl.ds(off, C)], midc)
                b2s = [plsc.load_gather(b2r, [jnp.full((16,), t + 1, jnp.int32)])
                       for t in range(8)]
                for i in range(C // 16):
                    sl = pl.ds(i * 16, 16)
                    sv = srcc[sl]
                    dv = dstc[sl]
                    mv = midc[sl]
                    lxs = plsc.load_gather(lxr, [sv])
                    lys = plsc.load_gather(lyr, [sv])
                    lxd = plsc.load_gather(lxr, [dv])
                    lyd = plsc.load_gather(lyr, [dv])
                    dx = lxd - lxs
                    dy = lyd - lys
                    d2 = dx * dx + dy * dy
                    bucket = jnp.zeros((16,), jnp.int32)
                    for t in range(8):
                        bucket = bucket + jnp.where(b2s[t] < d2, 1, 0)
                    sc_ = (plsc.load_gather(sd, [dv])
                           + plsc.load_gather(sm, [mv])
                           + plsc.load_gather(ss, [sv])
                           + plsc.load_gather(sdt, [bucket]))
                    beta = 1.0 / (1.0 + jnp.exp(-sc_))
                    cc2 = (plsc.load_gather(d0, [sv])
                           * plsc.load_gather(d0, [dv]))
                    av[sl] = beta * cc2
                    bv[sl] = cc2
                sync(av, a_hbm.at[pl.ds(off, C)])
                sync(bv, b_hbm.at[pl.ds(off, C)])
                return 0

            lax.fori_loop(0, NCHUNK, edgek, 0)

    return k(srcs, dsts, mids, sd_h, sm_h, ss_h, lx_h, ly_h, sdist_h,
             b2_h, p2_h)


def _sc_agg(h2pair, srcs, dsts, mids, a_h, b_h):
    mesh = plsc.VectorSubcoreMesh(core_axis_name="c", subcore_axis_name="s")

    @functools.partial(
        pl.kernel,
        out_type=jax.ShapeDtypeStruct((2, N_PAD, FH), jnp.float32),
        mesh=mesh,
        compiler_params=_SC_PARAMS,
        scratch_types=[
            pltpu.VMEM_SHARED((N_PAD, FH), jnp.float32),
            pltpu.VMEM((C,), jnp.int32),
            pltpu.VMEM((C,), jnp.int32),
            pltpu.VMEM((C,), jnp.int32),
            pltpu.VMEM((C,), jnp.int32),
            pltpu.VMEM((C,), jnp.int32),
            pltpu.VMEM((C,), jnp.float32),
            pltpu.VMEM((C,), jnp.float32),
            pltpu.VMEM((C, FH), jnp.float32),
            pltpu.VMEM((C, FH), jnp.float32),
            pltpu.SemaphoreType.DMA,
            pltpu.SemaphoreType.DMA,
        ],
    )
    def k(h2_hbm, src_hbm, dst_hbm, mid_hbm, a_hbm, b_hbm, out_hbm,
          acc, srcc, dstc, midc, isrc, imid, av, bv, hs, hm, sem1, sem2):
        c = lax.axis_index("c")
        s = lax.axis_index("s")
        zero16 = jnp.zeros((16,), jnp.float32)
        sync = pltpu.sync_copy

        def zrow(r, _):
            for j in range(FH // 16):
                hs[r, pl.ds(j * 16, 16)] = zero16
            return 0

        lax.fori_loop(0, C, zrow, 0)
        r0 = s * ROWS_PT
        for j in range(ROWS_PT // C):
            sync(hs, acc.at[pl.ds(r0 + j * C, C)])
        plsc.subcore_barrier()

        coff = (c * N).astype(jnp.int32)

        def edgek(kk, _):
            off = s * EPT + kk * C
            sync(src_hbm.at[pl.ds(off, C)], srcc)
            sync(dst_hbm.at[pl.ds(off, C)], dstc)
            sync(mid_hbm.at[pl.ds(off, C)], midc)
            sync(a_hbm.at[pl.ds(off, C)], av)
            sync(b_hbm.at[pl.ds(off, C)], bv)
            for i in range(C // 16):
                sl = pl.ds(i * 16, 16)
                isrc[sl] = srcc[sl] + coff
                imid[sl] = midc[sl] + coff
            cp1 = pltpu.async_copy(h2_hbm.at[isrc], hs, sem1)
            cp2 = pltpu.async_copy(h2_hbm.at[imid], hm, sem2)
            cp1.wait()
            cp2.wait()

            def row(e, _):
                es = jnp.full((16,), e, jnp.int32)
                ae = plsc.load_gather(av, [es])
                be = plsc.load_gather(bv, [es])
                for j in range(FH // 16):
                    csl = pl.ds(j * 16, 16)
                    hs[e, csl] = ae * hs[e, csl] + be * hm[e, csl]
                return 0

            lax.fori_loop(0, C, row, 0)
            sync(hs, acc.at[dstc], add=True)
            return 0

        lax.fori_loop(0, NCHUNK, edgek, 0)
        plsc.subcore_barrier()

        sync(acc.at[pl.ds(r0, ROWS_PT)], out_hbm.at[c, pl.ds(r0, ROWS_PT)])

    return k(h2pair, srcs, dsts, mids, a_h, b_h)


def kernel(feat, loc, edge_index, mid, W_fc2, W_fcd, W_w1, W_w2, vec_a,
           fin_table, boundaries):
    with jax.default_matmul_precision("highest"):
        a = vec_a[0]
        u_dst = W_w1[:, :F].T @ a
        u_dist = W_w1[:, F:].T @ a
        u_mid = W_w2[:, :F].T @ a
        u_src = W_w2[:, F:].T @ a
        v8 = jnp.zeros((F, 8), jnp.float32)
        v8 = v8.at[:, 0].set(W_fc2.T @ u_dst)
        v8 = v8.at[:, 1].set(W_fc2.T @ u_mid)
        v8 = v8.at[:, 2].set(W_fc2.T @ u_src)
        sdist_tbl = (fin_table @ W_fcd.T) @ u_dist
    sdist128 = jnp.zeros((128,), jnp.float32).at[:9].set(sdist_tbl)
    b2_128 = jnp.full((128,), 1e30, jnp.float32).at[1:9].set(
        boundaries * boundaries)
    p2_128 = jnp.zeros((128,), jnp.float32).at[:32].set(
        jnp.asarray([2.0 ** (-kk / 2.0) for kk in range(32)], jnp.float32))

    w_pair = W_fc2.reshape(2, FH, F)
    h2pair, st = _tc_dense(feat, w_pair, v8)

    srcs = edge_index[0]
    dsts = edge_index[1]
    a_h, b_h = _sc_coeffs(
        srcs, dsts, mid,
        st[:, 0] + 0.0, st[:, 1] + 0.0, st[:, 2] + 0.0,
        loc[:, 0] + 0.0, loc[:, 1] + 0.0,
        sdist128, b2_128, p2_128)
    out_pair = _sc_agg(h2pair, srcs, dsts, mid, a_h, b_h)
    return jnp.concatenate([out_pair[0, :N], out_pair[1, :N]], axis=1)


# speedup vs baseline: 5.1837x; 1.2953x over previous
import functools

import jax
import jax.numpy as jnp
from jax import lax
from jax.experimental import pallas as pl
from jax.experimental.pallas import tpu as pltpu, tpu_sc as plsc

N = 10000
E = 160000
F = 256
FH = 128
NT = 16
EPT = E // NT
C = 80
NCHUNK = EPT // C
N_PAD = 10240
ROWS_PT = N_PAD // NT
DEGP = NT * 640

_SC_PARAMS = pltpu.CompilerParams(needs_layout_passes=False)


def _tc_dense(feat, w_pair, v8):
    def body(f_ref, w_ref, v_ref, h2_ref, st_ref):
        f = f_ref[...]
        w = w_ref[0]
        h2_ref[...] = lax.dot_general(
            f, w, (((1,), (1,)), ((), ())),
            precision=lax.Precision.HIGHEST,
            preferred_element_type=jnp.float32)
        st_ref[...] = lax.dot_general(
            f, v_ref[...], (((1,), (0,)), ((), ())),
            precision=lax.Precision.HIGHEST,
            preferred_element_type=jnp.float32)

    return pl.pallas_call(
        body,
        grid=(2, 10),
        in_specs=[
            pl.BlockSpec((1000, 256), lambda j, i: (i, 0)),
            pl.BlockSpec((1, 128, 256), lambda j, i: (j, 0, 0)),
            pl.BlockSpec((256, 8), lambda j, i: (0, 0)),
        ],
        out_specs=[
            pl.BlockSpec((1000, 128), lambda j, i: (j * 10 + i, 0)),
            pl.BlockSpec((1000, 8), lambda j, i: (i, 0)),
        ],
        out_shape=[
            jax.ShapeDtypeStruct((2 * N, FH), jnp.float32),
            jax.ShapeDtypeStruct((N, 8), jnp.float32),
        ],
    )(feat, w_pair, v8)


def _sc_coeffs(srcs, dsts, mids, sd_h, sm_h, ss_h, lx_h, ly_h, sdist_h,
               b2_h, p2_h):
    mesh = plsc.VectorSubcoreMesh(core_axis_name="c", subcore_axis_name="s")

    @functools.partial(
        pl.kernel,
        out_type=jax.ShapeDtypeStruct((E * 8,), jnp.float32),
        mesh=mesh,
        compiler_params=_SC_PARAMS,
        scratch_types=[
            pltpu.VMEM_SHARED((DEGP,), jnp.float32),
            pltpu.VMEM((N,), jnp.float32),
            pltpu.VMEM((N,), jnp.float32),
            pltpu.VMEM((N,), jnp.float32),
            pltpu.VMEM((DEGP,), jnp.float32),
            pltpu.VMEM((N,), jnp.float32),
            pltpu.VMEM((N,), jnp.float32),
            pltpu.VMEM((128,), jnp.float32),
            pltpu.VMEM((128,), jnp.float32),
            pltpu.VMEM((128,), jnp.float32),
            pltpu.VMEM((C,), jnp.int32),
            pltpu.VMEM((C,), jnp.int32),
            pltpu.VMEM((C,), jnp.int32),
            pltpu.VMEM((C * 8,), jnp.float32),
            pltpu.VMEM((C,), jnp.float32),
            pltpu.VMEM((640,), jnp.float32),
        ],
    )
    def k(src_hbm, dst_hbm, mid_hbm, sd_hbm, sm_hbm, ss_hbm, lx_hbm,
          ly_hbm, sdist_hbm, b2_hbm, p2_hbm, ed_hbm,
          deg, sd, sm, ss, d0, lxr, lyr, sdt, b2r, p2r,
          srcc, dstc, midc, ebuf, ones, z640):
        c = lax.axis_index("c")
        s = lax.axis_index("s")
        zero16 = jnp.zeros((16,), jnp.float32)
        one16 = jnp.full((16,), 1.0, jnp.float32)
        sync = pltpu.sync_copy

        @pl.when(c == 0)
        def _core0():
            for i in range(C // 16):
                ones[pl.ds(i * 16, 16)] = one16

            def zb(i, _):
                z640[pl.ds(i * 16, 16)] = zero16
                return 0

            lax.fori_loop(0, 640 // 16, zb, 0)
            sync(z640, deg.at[pl.ds(s * 640, 640)])
            plsc.subcore_barrier()

            def degk(kk, _):
                off = s * EPT + kk * C
                sync(src_hbm.at[pl.ds(off, C)], srcc)
                sync(ones, deg.at[srcc], add=True)
                return 0

            lax.fori_loop(0, NCHUNK, degk, 0)
            plsc.subcore_barrier()

            sync(sd_hbm, sd)
            sync(sm_hbm, sm)
            sync(ss_hbm, ss)
            sync(lx_hbm, lxr)
            sync(ly_hbm, lyr)
            sync(sdist_hbm, sdt)
            sync(b2_hbm, b2r)
            sync(p2_hbm, p2r)
            sync(deg, d0)

            def rsq(i, _):
                sl = pl.ds(i * 16, 16)
                x = jnp.maximum(d0[sl], 1.0)
                kk = jnp.zeros((16,), jnp.int32)
                for t in range(1, 19):
                    kk = kk + jnp.where(x >= float(2 ** t), 1, 0)
                y = plsc.load_gather(p2r, [kk])
                for _ in range(5):
                    y = y * (1.5 - 0.5 * x * y * y)
                d0[sl] = y
                return 0

            lax.fori_loop(0, DEGP // 16, rsq, 0)

            def edgek(kk, _):
                off = s * EPT + kk * C
                sync(src_hbm.at[pl.ds(off, C)], srcc)
                sync(dst_hbm.at[pl.ds(off, C)], dstc)
                sync(mid_hbm.at[pl.ds(off, C)], midc)
                b2s = [plsc.load_gather(b2r, [jnp.full((16,), t + 1, jnp.int32)])
                       for t in range(8)]
                for i in range(C // 16):
                    sl = pl.ds(i * 16, 16)
                    sv = srcc[sl]
                    dv = dstc[sl]
                    mv = midc[sl]
                    lxs = plsc.load_gather(lxr, [sv])
                    lys = plsc.load_gather(lyr, [sv])
                    lxd = plsc.load_gather(lxr, [dv])
                    lyd = plsc.load_gather(lyr, [dv])
                    dx = lxd - lxs
                    dy = lyd - lys
                    d2 = dx * dx + dy * dy
                    bucket = jnp.zeros((16,), jnp.int32)
                    for t in range(8):
                        bucket = bucket + jnp.where(b2s[t] < d2, 1, 0)
                    sc_ = (plsc.load_gather(sd, [dv])
                           + plsc.load_gather(sm, [mv])
                           + plsc.load_gather(ss, [sv])
                           + plsc.load_gather(sdt, [bucket]))
                    beta = 1.0 / (1.0 + jnp.exp(-sc_))
                    cc2 = (plsc.load_gather(d0, [sv])
                           * plsc.load_gather(d0, [dv]))
                    rows8 = lax.iota(jnp.int32, 16) * 8 + (i * 128)
                    plsc.store_scatter(ebuf, [rows8], sv.astype(jnp.float32))
                    plsc.store_scatter(ebuf, [rows8 + 1],
                                       mv.astype(jnp.float32))
                    plsc.store_scatter(ebuf, [rows8 + 2],
                                       dv.astype(jnp.float32))
                    plsc.store_scatter(ebuf, [rows8 + 3], beta * cc2)
                    plsc.store_scatter(ebuf, [rows8 + 4], cc2)
                sync(ebuf, ed_hbm.at[pl.ds(off * 8, C * 8)])
                return 0

            lax.fori_loop(0, NCHUNK, edgek, 0)

    return k(srcs, dsts, mids, sd_h, sm_h, ss_h, lx_h, ly_h, sdist_h,
             b2_h, p2_h)


def _sc_agg(h2pair, ed_h):
    mesh = plsc.VectorSubcoreMesh(core_axis_name="c", subcore_axis_name="s")

    @functools.partial(
        pl.kernel,
        out_type=jax.ShapeDtypeStruct((2, N_PAD, FH), jnp.float32),
        mesh=mesh,
        compiler_params=_SC_PARAMS,
        scratch_types=[
            pltpu.VMEM_SHARED((N_PAD, FH), jnp.float32),
            pltpu.VMEM((C * 8,), jnp.float32),
            pltpu.VMEM((C * 8,), jnp.float32),
            pltpu.VMEM((C,), jnp.int32),
            pltpu.VMEM((C,), jnp.int32),
            pltpu.VMEM((C,), jnp.int32),
            pltpu.VMEM((C,), jnp.int32),
            pltpu.VMEM((C,), jnp.int32),
            pltpu.VMEM((C,), jnp.int32),
            pltpu.VMEM((C, FH), jnp.float32),
            pltpu.VMEM((C, FH), jnp.float32),
            pltpu.VMEM((C, FH), jnp.float32),
            pltpu.VMEM((C, FH), jnp.float32),
            pltpu.SemaphoreType.DMA,
            pltpu.SemaphoreType.DMA,
            pltpu.SemaphoreType.DMA,
            pltpu.SemaphoreType.DMA,
        ],
    )
    def k(h2_hbm, ed_hbm, out_hbm, acc,
          eb0, eb1, is0, is1, im0, im1, ds0, ds1,
          hs0, hs1, hm0, hm1, sg0, sg1, ss0, ss1):
        c = lax.axis_index("c")
        s = lax.axis_index("s")
        zero16 = jnp.zeros((16,), jnp.float32)
        sync = pltpu.sync_copy
        eb = (eb0, eb1)
        isr = (is0, is1)
        imr = (im0, im1)
        dsr = (ds0, ds1)
        hs = (hs0, hs1)
        hm = (hm0, hm1)
        sg = (sg0, sg1)
        ssem = (ss0, ss1)

        def zrow(r, _):
            for j in range(FH // 16):
                hs0[r, pl.ds(j * 16, 16)] = zero16
            return 0

        lax.fori_loop(0, C, zrow, 0)
        r0 = s * ROWS_PT
        for j in range(ROWS_PT // C):
            sync(hs0, acc.at[pl.ds(r0 + j * C, C)])
        plsc.subcore_barrier()

        coff = (c * N).astype(jnp.int32)
        ebase = s * EPT

        def prefetch(b, kk):
            off = ebase + kk * C
            sync(ed_hbm.at[pl.ds(off * 8, C * 8)], eb[b])
            for i in range(C // 16):
                sl = pl.ds(i * 16, 16)
                rows8 = lax.iota(jnp.int32, 16) * 8 + (i * 128)
                sv = plsc.load_gather(eb[b], [rows8]).astype(jnp.int32)
                mv = plsc.load_gather(eb[b], [rows8 + 1]).astype(jnp.int32)
                dv = plsc.load_gather(eb[b], [rows8 + 2]).astype(jnp.int32)
                isr[b][sl] = sv + coff
                imr[b][sl] = mv + coff
                dsr[b][sl] = dv
            cps = pltpu.async_copy(h2_hbm.at[isr[b]], hs[b], sg[b])
            cpm = pltpu.async_copy(h2_hbm.at[imr[b]], hm[b], sg[b])
            return cps, cpm

        def consume(b, cps, cpm):
            cps.wait()
            cpm.wait()

            def row(e, _):
                ae = plsc.load_gather(eb[b], [jnp.full((16,), e * 8 + 3,
                                                       jnp.int32)])
                be = plsc.load_gather(eb[b], [jnp.full((16,), e * 8 + 4,
                                                       jnp.int32)])
                for j in range(FH // 16):
                    csl = pl.ds(j * 16, 16)
                    hs[b][e, csl] = ae * hs[b][e, csl] + be * hm[b][e, csl]
                return 0

            lax.fori_loop(0, C, row, 0)
            return pltpu.async_copy(hs[b], acc.at[dsr[b]], ssem[b],
                                    add=True)

        def body(g, _):
            k0 = 2 * g
            k1 = 2 * g + 1
            cps0, cpm0 = prefetch(0, k0)

            @pl.when(k1 < NCHUNK)
            def _full():
                cps1, cpm1 = prefetch(1, k1)
                sc0 = consume(0, cps0, cpm0)
                sc1 = consume(1, cps1, cpm1)
                sc0.wait()
                sc1.wait()

            @pl.when(k1 >= NCHUNK)
            def _tail():
                sc0 = consume(0, cps0, cpm0)
                sc0.wait()

            return 0

        lax.fori_loop(0, (NCHUNK + 1) // 2, body, 0)
        plsc.subcore_barrier()

        sync(acc.at[pl.ds(r0, ROWS_PT)], out_hbm.at[c, pl.ds(r0, ROWS_PT)])

    return k(h2pair, ed_h)


def kernel(feat, loc, edge_index, mid, W_fc2, W_fcd, W_w1, W_w2, vec_a,
           fin_table, boundaries):
    with jax.default_matmul_precision("highest"):
        a = vec_a[0]
        u_dst = W_w1[:, :F].T @ a
        u_dist = W_w1[:, F:].T @ a
        u_mid = W_w2[:, :F].T @ a
        u_src = W_w2[:, F:].T @ a
        v8 = jnp.zeros((F, 8), jnp.float32)
        v8 = v8.at[:, 0].set(W_fc2.T @ u_dst)
        v8 = v8.at[:, 1].set(W_fc2.T @ u_mid)
        v8 = v8.at[:, 2].set(W_fc2.T @ u_src)
        sdist_tbl = (fin_table @ W_fcd.T) @ u_dist
    sdist128 = jnp.zeros((128,), jnp.float32).at[:9].set(sdist_tbl)
    b2_128 = jnp.full((128,), 1e30, jnp.float32).at[1:9].set(
        boundaries * boundaries)
    p2_128 = jnp.zeros((128,), jnp.float32).at[:32].set(
        jnp.asarray([2.0 ** (-kk / 2.0) for kk in range(32)], jnp.float32))

    w_pair = W_fc2.reshape(2, FH, F)
    h2pair, st = _tc_dense(feat, w_pair, v8)

    srcs = edge_index[0]
    dsts = edge_index[1]
    ed_h = _sc_coeffs(
        srcs, dsts, mid,
        st[:, 0] + 0.0, st[:, 1] + 0.0, st[:, 2] + 0.0,
        loc[:, 0] + 0.0, loc[:, 1] + 0.0,
        sdist128, b2_128, p2_128)
    out_pair = _sc_agg(h2pair, ed_h)
    return jnp.concatenate([out_pair[0, :N], out_pair[1, :N]], axis=1)


# speedup vs baseline: 5.6903x; 1.0977x over previous
import functools

import jax
import jax.numpy as jnp
from jax import lax
from jax.experimental import pallas as pl
from jax.experimental.pallas import tpu as pltpu, tpu_sc as plsc

N = 10000
E = 160000
F = 256
FH = 128
NT = 16
EPT = E // NT
EPW = E // (2 * NT)
C = 80
NCHUNK = EPT // C
N_PAD = 10240
ROWS_PT = N_PAD // NT
DEGP = NT * 640

_SC_PARAMS = pltpu.CompilerParams(needs_layout_passes=False)


def _tc_dense(feat, w_pair, v8):
    def body(f_ref, w_ref, v_ref, h2_ref, st_ref):
        f = f_ref[...]
        w = w_ref[0]
        h2_ref[...] = lax.dot_general(
            f, w, (((1,), (1,)), ((), ())),
            precision=lax.Precision.HIGHEST,
            preferred_element_type=jnp.float32)
        st_ref[...] = lax.dot_general(
            f, v_ref[...], (((1,), (0,)), ((), ())),
            precision=lax.Precision.HIGHEST,
            preferred_element_type=jnp.float32)

    return pl.pallas_call(
        body,
        grid=(2, 10),
        in_specs=[
            pl.BlockSpec((1000, 256), lambda j, i: (i, 0)),
            pl.BlockSpec((1, 128, 256), lambda j, i: (j, 0, 0)),
            pl.BlockSpec((256, 8), lambda j, i: (0, 0)),
        ],
        out_specs=[
            pl.BlockSpec((1000, 128), lambda j, i: (j * 10 + i, 0)),
            pl.BlockSpec((1000, 8), lambda j, i: (i, 0)),
        ],
        out_shape=[
            jax.ShapeDtypeStruct((2 * N, FH), jnp.float32),
            jax.ShapeDtypeStruct((N, 8), jnp.float32),
        ],
    )(feat, w_pair, v8)


def _sc_coeffs(srcs, dsts, mids, sd_h, sm_h, ss_h, lx_h, ly_h, sdist_h,
               b2_h, p2_h):
    mesh = plsc.VectorSubcoreMesh(core_axis_name="c", subcore_axis_name="s")

    @functools.partial(
        pl.kernel,
        out_type=jax.ShapeDtypeStruct((E * 8,), jnp.float32),
        mesh=mesh,
        compiler_params=_SC_PARAMS,
        scratch_types=[
            pltpu.VMEM_SHARED((DEGP,), jnp.float32),
            pltpu.VMEM((N,), jnp.float32),
            pltpu.VMEM((N,), jnp.float32),
            pltpu.VMEM((N,), jnp.float32),
            pltpu.VMEM((DEGP,), jnp.float32),
            pltpu.VMEM((N,), jnp.float32),
            pltpu.VMEM((N,), jnp.float32),
            pltpu.VMEM((128,), jnp.float32),
            pltpu.VMEM((128,), jnp.float32),
            pltpu.VMEM((128,), jnp.float32),
            pltpu.VMEM((C,), jnp.int32),
            pltpu.VMEM((C,), jnp.int32),
            pltpu.VMEM((C,), jnp.int32),
            pltpu.VMEM((C * 8,), jnp.float32),
            pltpu.VMEM((C,), jnp.float32),
            pltpu.VMEM((640,), jnp.float32),
        ],
    )
    def k(src_hbm, dst_hbm, mid_hbm, sd_hbm, sm_hbm, ss_hbm, lx_hbm,
          ly_hbm, sdist_hbm, b2_hbm, p2_hbm, ed_hbm,
          deg, sd, sm, ss, d0, lxr, lyr, sdt, b2r, p2r,
          srcc, dstc, midc, ebuf, ones, z640):
        c = lax.axis_index("c")
        s = lax.axis_index("s")
        zero16 = jnp.zeros((16,), jnp.float32)
        one16 = jnp.full((16,), 1.0, jnp.float32)
        sync = pltpu.sync_copy

        for i in range(C // 16):
            ones[pl.ds(i * 16, 16)] = one16

        def zb(i, _):
            z640[pl.ds(i * 16, 16)] = zero16
            return 0

        lax.fori_loop(0, 640 // 16, zb, 0)
        sync(z640, deg.at[pl.ds(s * 640, 640)])
        plsc.subcore_barrier()

        def degk(kk, _):
            off = s * EPT + kk * C
            sync(src_hbm.at[pl.ds(off, C)], srcc)
            sync(ones, deg.at[srcc], add=True)
            return 0

        lax.fori_loop(0, NCHUNK, degk, 0)
        plsc.subcore_barrier()

        sync(sd_hbm, sd)
        sync(sm_hbm, sm)
        sync(ss_hbm, ss)
        sync(lx_hbm, lxr)
        sync(ly_hbm, lyr)
        sync(sdist_hbm, sdt)
        sync(b2_hbm, b2r)
        sync(p2_hbm, p2r)
        sync(deg, d0)

        def rsq(i, _):
            sl = pl.ds(i * 16, 16)
            x = jnp.maximum(d0[sl], 1.0)
            kk = jnp.zeros((16,), jnp.int32)
            for t in range(1, 19):
                kk = kk + jnp.where(x >= float(2 ** t), 1, 0)
            y = plsc.load_gather(p2r, [kk])
            for _ in range(5):
                y = y * (1.5 - 0.5 * x * y * y)
            d0[sl] = y
            return 0

        lax.fori_loop(0, DEGP // 16, rsq, 0)

        wbase = (s * 2 + c) * EPW

        def coeff_chunk(off):
            sync(src_hbm.at[pl.ds(off, C)], srcc)
            sync(dst_hbm.at[pl.ds(off, C)], dstc)
            sync(mid_hbm.at[pl.ds(off, C)], midc)
            b2s = [plsc.load_gather(b2r, [jnp.full((16,), t + 1, jnp.int32)])
                   for t in range(8)]
            for i in range(C // 16):
                sl = pl.ds(i * 16, 16)
                sv = srcc[sl]
                dv = dstc[sl]
                mv = midc[sl]
                lxs = plsc.load_gather(lxr, [sv])
                lys = plsc.load_gather(lyr, [sv])
                lxd = plsc.load_gather(lxr, [dv])
                lyd = plsc.load_gather(lyr, [dv])
                dx = lxd - lxs
                dy = lyd - lys
                d2 = dx * dx + dy * dy
                bucket = jnp.zeros((16,), jnp.int32)
                for t in range(8):
                    bucket = bucket + jnp.where(b2s[t] < d2, 1, 0)
                sc_ = (plsc.load_gather(sd, [dv])
                       + plsc.load_gather(sm, [mv])
                       + plsc.load_gather(ss, [sv])
                       + plsc.load_gather(sdt, [bucket]))
                beta = 1.0 / (1.0 + jnp.exp(-sc_))
                cc2 = (plsc.load_gather(d0, [sv])
                       * plsc.load_gather(d0, [dv]))
                rows8 = lax.iota(jnp.int32, 16) * 8 + (i * 128)
                plsc.store_scatter(ebuf, [rows8], sv.astype(jnp.float32))
                plsc.store_scatter(ebuf, [rows8 + 1],
                                   mv.astype(jnp.float32))
                plsc.store_scatter(ebuf, [rows8 + 2],
                                   dv.astype(jnp.float32))
                plsc.store_scatter(ebuf, [rows8 + 3], beta * cc2)
                plsc.store_scatter(ebuf, [rows8 + 4], cc2)
            sync(ebuf, ed_hbm.at[pl.ds(off * 8, C * 8)])

        def edgek(kk, _):
            coeff_chunk(wbase + kk * C)
            return 0

        lax.fori_loop(0, EPW // C, edgek, 0)
        coeff_chunk(wbase + EPW - C)

    return k(srcs, dsts, mids, sd_h, sm_h, ss_h, lx_h, ly_h, sdist_h,
             b2_h, p2_h)


def _sc_agg(h2pair, ed_h):
    mesh = plsc.VectorSubcoreMesh(core_axis_name="c", subcore_axis_name="s")

    @functools.partial(
        pl.kernel,
        out_type=jax.ShapeDtypeStruct((2, N_PAD, FH), jnp.float32),
        mesh=mesh,
        compiler_params=_SC_PARAMS,
        scratch_types=[
            pltpu.VMEM_SHARED((N_PAD, FH), jnp.float32),
            pltpu.VMEM((C * 8,), jnp.float32),
            pltpu.VMEM((C * 8,), jnp.float32),
            pltpu.VMEM((C,), jnp.int32),
            pltpu.VMEM((C,), jnp.int32),
            pltpu.VMEM((C,), jnp.int32),
            pltpu.VMEM((C,), jnp.int32),
            pltpu.VMEM((C,), jnp.int32),
            pltpu.VMEM((C,), jnp.int32),
            pltpu.VMEM((C, FH), jnp.float32),
            pltpu.VMEM((C, FH), jnp.float32),
            pltpu.VMEM((C, FH), jnp.float32),
            pltpu.VMEM((C, FH), jnp.float32),
            pltpu.SemaphoreType.DMA,
            pltpu.SemaphoreType.DMA,
            pltpu.SemaphoreType.DMA,
            pltpu.SemaphoreType.DMA,
        ],
    )
    def k(h2_hbm, ed_hbm, out_hbm, acc,
          eb0, eb1, is0, is1, im0, im1, ds0, ds1,
          hs0, hs1, hm0, hm1, sg0, sg1, ss0, ss1):
        c = lax.axis_index("c")
        s = lax.axis_index("s")
        zero16 = jnp.zeros((16,), jnp.float32)
        sync = pltpu.sync_copy
        eb = (eb0, eb1)
        isr = (is0, is1)
        imr = (im0, im1)
        dsr = (ds0, ds1)
        hs = (hs0, hs1)
        hm = (hm0, hm1)
        sg = (sg0, sg1)
        ssem = (ss0, ss1)

        def zrow(r, _):
            for j in range(FH // 16):
                hs0[r, pl.ds(j * 16, 16)] = zero16
            return 0

        lax.fori_loop(0, C, zrow, 0)
        r0 = s * ROWS_PT
        for j in range(ROWS_PT // C):
            sync(hs0, acc.at[pl.ds(r0 + j * C, C)])
        plsc.subcore_barrier()

        coff = (c * N).astype(jnp.int32)
        ebase = s * EPT

        def prefetch(b, kk):
            off = ebase + kk * C
            sync(ed_hbm.at[pl.ds(off * 8, C * 8)], eb[b])
            for i in range(C // 16):
                sl = pl.ds(i * 16, 16)
                rows8 = lax.iota(jnp.int32, 16) * 8 + (i * 128)
                sv = plsc.load_gather(eb[b], [rows8]).astype(jnp.int32)
                mv = plsc.load_gather(eb[b], [rows8 + 1]).astype(jnp.int32)
                dv = plsc.load_gather(eb[b], [rows8 + 2]).astype(jnp.int32)
                isr[b][sl] = sv + coff
                imr[b][sl] = mv + coff
                dsr[b][sl] = dv
            cps = pltpu.async_copy(h2_hbm.at[isr[b]], hs[b], sg[b])
            cpm = pltpu.async_copy(h2_hbm.at[imr[b]], hm[b], sg[b])
            return cps, cpm

        def consume(b, cps, cpm):
            cps.wait()
            cpm.wait()

            def row(e, _):
                ae = plsc.load_gather(eb[b], [jnp.full((16,), e * 8 + 3,
                                                       jnp.int32)])
                be = plsc.load_gather(eb[b], [jnp.full((16,), e * 8 + 4,
                                                       jnp.int32)])
                for j in range(FH // 16):
                    csl = pl.ds(j * 16, 16)
                    hs[b][e, csl] = ae * hs[b][e, csl] + be * hm[b][e, csl]
                return 0

            lax.fori_loop(0, C, row, 0)
            return pltpu.async_copy(hs[b], acc.at[dsr[b]], ssem[b],
                                    add=True)

        def body(g, _):
            k0 = 2 * g
            k1 = 2 * g + 1
            cps0, cpm0 = prefetch(0, k0)

            @pl.when(k1 < NCHUNK)
            def _full():
                cps1, cpm1 = prefetch(1, k1)
                sc0 = consume(0, cps0, cpm0)
                sc1 = consume(1, cps1, cpm1)
                sc0.wait()
                sc1.wait()

            @pl.when(k1 >= NCHUNK)
            def _tail():
                sc0 = consume(0, cps0, cpm0)
                sc0.wait()

            return 0

        lax.fori_loop(0, (NCHUNK + 1) // 2, body, 0)
        plsc.subcore_barrier()

        sync(acc.at[pl.ds(r0, ROWS_PT)], out_hbm.at[c, pl.ds(r0, ROWS_PT)])

    return k(h2pair, ed_h)


def kernel(feat, loc, edge_index, mid, W_fc2, W_fcd, W_w1, W_w2, vec_a,
           fin_table, boundaries):
    with jax.default_matmul_precision("highest"):
        a = vec_a[0]
        u_dst = W_w1[:, :F].T @ a
        u_dist = W_w1[:, F:].T @ a
        u_mid = W_w2[:, :F].T @ a
        u_src = W_w2[:, F:].T @ a
        v8 = jnp.zeros((F, 8), jnp.float32)
        v8 = v8.at[:, 0].set(W_fc2.T @ u_dst)
        v8 = v8.at[:, 1].set(W_fc2.T @ u_mid)
        v8 = v8.at[:, 2].set(W_fc2.T @ u_src)
        sdist_tbl = (fin_table @ W_fcd.T) @ u_dist
    sdist128 = jnp.zeros((128,), jnp.float32).at[:9].set(sdist_tbl)
    b2_128 = jnp.full((128,), 1e30, jnp.float32).at[1:9].set(
        boundaries * boundaries)
    p2_128 = jnp.zeros((128,), jnp.float32).at[:32].set(
        jnp.asarray([2.0 ** (-kk / 2.0) for kk in range(32)], jnp.float32))

    w_pair = W_fc2.reshape(2, FH, F)
    h2pair, st = _tc_dense(feat, w_pair, v8)

    srcs = edge_index[0]
    dsts = edge_index[1]
    ed_h = _sc_coeffs(
        srcs, dsts, mid,
        st[:, 0] + 0.0, st[:, 1] + 0.0, st[:, 2] + 0.0,
        loc[:, 0] + 0.0, loc[:, 1] + 0.0,
        sdist128, b2_128, p2_128)
    out_pair = _sc_agg(h2pair, ed_h)
    return jnp.concatenate([out_pair[0, :N], out_pair[1, :N]], axis=1)


# speedup vs baseline: 5.9847x; 1.0517x over previous
import functools

import jax
import jax.numpy as jnp
from jax import lax
from jax.experimental import pallas as pl
from jax.experimental.pallas import tpu as pltpu, tpu_sc as plsc

N = 10000
E = 160000
F = 256
FH = 128
NT = 16
EPT = E // NT
EPW = E // (2 * NT)
C = 80
NCHUNK = EPT // C
N_PAD = 10240
ROWS_PT = N_PAD // NT
DEGP = NT * 640

_SC_PARAMS = pltpu.CompilerParams(needs_layout_passes=False)


def _tc_dense(feat, w_pair, v8):
    def body(f_ref, w_ref, v_ref, h2_ref, st_ref):
        f = f_ref[...]
        w = w_ref[0]
        h2_ref[...] = lax.dot_general(
            f, w, (((1,), (1,)), ((), ())),
            precision=lax.Precision.HIGHEST,
            preferred_element_type=jnp.float32)
        st_ref[...] = lax.dot_general(
            f, v_ref[...], (((1,), (0,)), ((), ())),
            precision=lax.Precision.HIGHEST,
            preferred_element_type=jnp.float32)

    return pl.pallas_call(
        body,
        grid=(2, 10),
        in_specs=[
            pl.BlockSpec((1000, 256), lambda j, i: (i, 0)),
            pl.BlockSpec((1, 128, 256), lambda j, i: (j, 0, 0)),
            pl.BlockSpec((256, 8), lambda j, i: (0, 0)),
        ],
        out_specs=[
            pl.BlockSpec((1000, 128), lambda j, i: (j * 10 + i, 0)),
            pl.BlockSpec((1000, 8), lambda j, i: (i, 0)),
        ],
        out_shape=[
            jax.ShapeDtypeStruct((2 * N, FH), jnp.float32),
            jax.ShapeDtypeStruct((N, 8), jnp.float32),
        ],
    )(feat, w_pair, v8)


def _sc_coeffs(srcs, dsts, mids, sd_h, sm_h, ss_h, lx_h, ly_h, sdist_h,
               b2_h, p2_h):
    mesh = plsc.VectorSubcoreMesh(core_axis_name="c", subcore_axis_name="s")

    @functools.partial(
        pl.kernel,
        out_type=jax.ShapeDtypeStruct((E * 8,), jnp.float32),
        mesh=mesh,
        compiler_params=_SC_PARAMS,
        scratch_types=[
            pltpu.VMEM_SHARED((DEGP,), jnp.float32),
            pltpu.VMEM((N,), jnp.float32),
            pltpu.VMEM((N,), jnp.float32),
            pltpu.VMEM((N,), jnp.float32),
            pltpu.VMEM((DEGP,), jnp.float32),
            pltpu.VMEM((N,), jnp.float32),
            pltpu.VMEM((N,), jnp.float32),
            pltpu.VMEM((128,), jnp.float32),
            pltpu.VMEM((128,), jnp.float32),
            pltpu.VMEM((128,), jnp.float32),
            pltpu.VMEM((C,), jnp.int32),
            pltpu.VMEM((C,), jnp.int32),
            pltpu.VMEM((C,), jnp.int32),
            pltpu.VMEM((C * 8,), jnp.float32),
            pltpu.VMEM((C,), jnp.float32),
            pltpu.VMEM((640,), jnp.float32),
            pltpu.VMEM((2000,), jnp.int32),
        ],
    )
    def k(src_hbm, dst_hbm, mid_hbm, sd_hbm, sm_hbm, ss_hbm, lx_hbm,
          ly_hbm, sdist_hbm, b2_hbm, p2_hbm, ed_hbm,
          deg, sd, sm, ss, d0, lxr, lyr, sdt, b2r, p2r,
          srcc, dstc, midc, ebuf, ones, z640, sstg):
        c = lax.axis_index("c")
        s = lax.axis_index("s")
        zero16 = jnp.zeros((16,), jnp.float32)
        one16 = jnp.full((16,), 1.0, jnp.float32)
        sync = pltpu.sync_copy

        for i in range(C // 16):
            ones[pl.ds(i * 16, 16)] = one16

        def zb(i, _):
            z640[pl.ds(i * 16, 16)] = zero16
            return 0

        lax.fori_loop(0, 640 // 16, zb, 0)
        sync(z640, deg.at[pl.ds(s * 640, 640)])
        plsc.subcore_barrier()

        def degblk(blk, _):
            sync(src_hbm.at[pl.ds(s * EPT + blk * 2000, 2000)], sstg)

            def degk(kk, _):
                for i in range(C // 16):
                    srcc[pl.ds(i * 16, 16)] = sstg[pl.ds(kk * C + i * 16, 16)]
                sync(ones, deg.at[srcc], add=True)
                return 0

            lax.fori_loop(0, 2000 // C, degk, 0)
            return 0

        lax.fori_loop(0, EPT // 2000, degblk, 0)
        plsc.subcore_barrier()

        sync(sd_hbm, sd)
        sync(sm_hbm, sm)
        sync(ss_hbm, ss)
        sync(lx_hbm, lxr)
        sync(ly_hbm, lyr)
        sync(sdist_hbm, sdt)
        sync(b2_hbm, b2r)
        sync(p2_hbm, p2r)
        sync(deg, d0)

        def rsq(i, _):
            sl = pl.ds(i * 16, 16)
            x = jnp.maximum(d0[sl], 1.0)
            kk = jnp.zeros((16,), jnp.int32)
            for t in range(1, 19):
                kk = kk + jnp.where(x >= float(2 ** t), 1, 0)
            y = plsc.load_gather(p2r, [kk])
            for _ in range(5):
                y = y * (1.5 - 0.5 * x * y * y)
            d0[sl] = y
            return 0

        lax.fori_loop(0, DEGP // 16, rsq, 0)

        wbase = (s * 2 + c) * EPW

        def coeff_chunk(off):
            sync(src_hbm.at[pl.ds(off, C)], srcc)
            sync(dst_hbm.at[pl.ds(off, C)], dstc)
            sync(mid_hbm.at[pl.ds(off, C)], midc)
            b2s = [plsc.load_gather(b2r, [jnp.full((16,), t + 1, jnp.int32)])
                   for t in range(8)]
            for i in range(C // 16):
                sl = pl.ds(i * 16, 16)
                sv = srcc[sl]
                dv = dstc[sl]
                mv = midc[sl]
                lxs = plsc.load_gather(lxr, [sv])
                lys = plsc.load_gather(lyr, [sv])
                lxd = plsc.load_gather(lxr, [dv])
                lyd = plsc.load_gather(lyr, [dv])
                dx = lxd - lxs
                dy = lyd - lys
                d2 = dx * dx + dy * dy
                bucket = jnp.zeros((16,), jnp.int32)
                for t in range(8):
                    bucket = bucket + jnp.where(b2s[t] < d2, 1, 0)
                sc_ = (plsc.load_gather(sd, [dv])
                       + plsc.load_gather(sm, [mv])
                       + plsc.load_gather(ss, [sv])
                       + plsc.load_gather(sdt, [bucket]))
                beta = 1.0 / (1.0 + jnp.exp(-sc_))
                cc2 = (plsc.load_gather(d0, [sv])
                       * plsc.load_gather(d0, [dv]))
                rows8 = lax.iota(jnp.int32, 16) * 8 + (i * 128)
                plsc.store_scatter(ebuf, [rows8], sv.astype(jnp.float32))
                plsc.store_scatter(ebuf, [rows8 + 1],
                                   mv.astype(jnp.float32))
                plsc.store_scatter(ebuf, [rows8 + 2],
                                   dv.astype(jnp.float32))
                plsc.store_scatter(ebuf, [rows8 + 3], beta * cc2)
                plsc.store_scatter(ebuf, [rows8 + 4], cc2)
            sync(ebuf, ed_hbm.at[pl.ds(off * 8, C * 8)])

        def edgek(kk, _):
            coeff_chunk(wbase + kk * C)
            return 0

        lax.fori_loop(0, EPW // C, edgek, 0)
        coeff_chunk(wbase + EPW - C)

    return k(srcs, dsts, mids, sd_h, sm_h, ss_h, lx_h, ly_h, sdist_h,
             b2_h, p2_h)


def _sc_agg(h2pair, ed_h):
    mesh = plsc.VectorSubcoreMesh(core_axis_name="c", subcore_axis_name="s")

    @functools.partial(
        pl.kernel,
        out_type=jax.ShapeDtypeStruct((2, N_PAD, FH), jnp.float32),
        mesh=mesh,
        compiler_params=_SC_PARAMS,
        scratch_types=[
            pltpu.VMEM_SHARED((N_PAD, FH), jnp.float32),
            pltpu.VMEM((C * 8,), jnp.float32),
            pltpu.VMEM((C * 8,), jnp.float32),
            pltpu.VMEM((C,), jnp.int32),
            pltpu.VMEM((C,), jnp.int32),
            pltpu.VMEM((C,), jnp.int32),
            pltpu.VMEM((C,), jnp.int32),
            pltpu.VMEM((C,), jnp.int32),
            pltpu.VMEM((C,), jnp.int32),
            pltpu.VMEM((C, FH), jnp.float32),
            pltpu.VMEM((C, FH), jnp.float32),
            pltpu.VMEM((C, FH), jnp.float32),
            pltpu.VMEM((C, FH), jnp.float32),
            pltpu.SemaphoreType.DMA,
            pltpu.SemaphoreType.DMA,
            pltpu.SemaphoreType.DMA,
            pltpu.SemaphoreType.DMA,
        ],
    )
    def k(h2_hbm, ed_hbm, out_hbm, acc,
          eb0, eb1, is0, is1, im0, im1, ds0, ds1,
          hs0, hs1, hm0, hm1, sg0, sg1, ss0, ss1):
        c = lax.axis_index("c")
        s = lax.axis_index("s")
        zero16 = jnp.zeros((16,), jnp.float32)
        sync = pltpu.sync_copy
        eb = (eb0, eb1)
        isr = (is0, is1)
        imr = (im0, im1)
        dsr = (ds0, ds1)
        hs = (hs0, hs1)
        hm = (hm0, hm1)
        sg = (sg0, sg1)
        ssem = (ss0, ss1)

        def zrow(r, _):
            for j in range(FH // 16):
                hs0[r, pl.ds(j * 16, 16)] = zero16
            return 0

        lax.fori_loop(0, C, zrow, 0)
        r0 = s * ROWS_PT
        for j in range(ROWS_PT // C):
            sync(hs0, acc.at[pl.ds(r0 + j * C, C)])
        plsc.subcore_barrier()

        coff = (c * N).astype(jnp.int32)
        ebase = s * EPT

        def prefetch(b, kk):
            off = ebase + kk * C
            sync(ed_hbm.at[pl.ds(off * 8, C * 8)], eb[b])
            for i in range(C // 16):
                sl = pl.ds(i * 16, 16)
                rows8 = lax.iota(jnp.int32, 16) * 8 + (i * 128)
                sv = plsc.load_gather(eb[b], [rows8]).astype(jnp.int32)
                mv = plsc.load_gather(eb[b], [rows8 + 1]).astype(jnp.int32)
                dv = plsc.load_gather(eb[b], [rows8 + 2]).astype(jnp.int32)
                isr[b][sl] = sv + coff
                imr[b][sl] = mv + coff
                dsr[b][sl] = dv
            cps = pltpu.async_copy(h2_hbm.at[isr[b]], hs[b], sg[b])
            cpm = pltpu.async_copy(h2_hbm.at[imr[b]], hm[b], sg[b])
            return cps, cpm

        def consume(b, cps, cpm):
            cps.wait()
            cpm.wait()

            def row(e, _):
                ae = plsc.load_gather(eb[b], [jnp.full((16,), e * 8 + 3,
                                                       jnp.int32)])
                be = plsc.load_gather(eb[b], [jnp.full((16,), e * 8 + 4,
                                                       jnp.int32)])
                for j in range(FH // 16):
                    csl = pl.ds(j * 16, 16)
                    hs[b][e, csl] = ae * hs[b][e, csl] + be * hm[b][e, csl]
                return 0

            lax.fori_loop(0, C, row, 0)
            return pltpu.async_copy(hs[b], acc.at[dsr[b]], ssem[b],
                                    add=True)

        def body(g, _):
            k0 = 2 * g
            k1 = 2 * g + 1
            cps0, cpm0 = prefetch(0, k0)

            @pl.when(k1 < NCHUNK)
            def _full():
                cps1, cpm1 = prefetch(1, k1)
                sc0 = consume(0, cps0, cpm0)
                sc1 = consume(1, cps1, cpm1)
                sc0.wait()
                sc1.wait()

            @pl.when(k1 >= NCHUNK)
            def _tail():
                sc0 = consume(0, cps0, cpm0)
                sc0.wait()

            return 0

        lax.fori_loop(0, (NCHUNK + 1) // 2, body, 0)
        plsc.subcore_barrier()

        sync(acc.at[pl.ds(r0, ROWS_PT)], out_hbm.at[c, pl.ds(r0, ROWS_PT)])

    return k(h2pair, ed_h)


def kernel(feat, loc, edge_index, mid, W_fc2, W_fcd, W_w1, W_w2, vec_a,
           fin_table, boundaries):
    with jax.default_matmul_precision("highest"):
        a = vec_a[0]
        u_dst = W_w1[:, :F].T @ a
        u_dist = W_w1[:, F:].T @ a
        u_mid = W_w2[:, :F].T @ a
        u_src = W_w2[:, F:].T @ a
        v8 = jnp.zeros((F, 8), jnp.float32)
        v8 = v8.at[:, 0].set(W_fc2.T @ u_dst)
        v8 = v8.at[:, 1].set(W_fc2.T @ u_mid)
        v8 = v8.at[:, 2].set(W_fc2.T @ u_src)
        sdist_tbl = (fin_table @ W_fcd.T) @ u_dist
    sdist128 = jnp.zeros((128,), jnp.float32).at[:9].set(sdist_tbl)
    b2_128 = jnp.full((128,), 1e30, jnp.float32).at[1:9].set(
        boundaries * boundaries)
    p2_128 = jnp.zeros((128,), jnp.float32).at[:32].set(
        jnp.asarray([2.0 ** (-kk / 2.0) for kk in range(32)], jnp.float32))

    w_pair = W_fc2.reshape(2, FH, F)
    h2pair, st = _tc_dense(feat, w_pair, v8)

    srcs = edge_index[0]
    dsts = edge_index[1]
    ed_h = _sc_coeffs(
        srcs, dsts, mid,
        st[:, 0] + 0.0, st[:, 1] + 0.0, st[:, 2] + 0.0,
        loc[:, 0] + 0.0, loc[:, 1] + 0.0,
        sdist128, b2_128, p2_128)
    out_pair = _sc_agg(h2pair, ed_h)
    return jnp.concatenate([out_pair[0, :N], out_pair[1, :N]], axis=1)
